# baseline (speedup 1.0000x reference)
import functools

import jax
import jax.numpy as jnp
from jax import lax
from jax.experimental import pallas as pl
from jax.experimental.pallas import tpu as pltpu
from jax.experimental.pallas import tpu_sc as plsc


def _sc_gather(table, idx_flat):
    n = idx_flat.shape[0]
    e = table.shape[1]
    info = plsc.get_sparse_core_info()
    nw = info.num_cores * info.num_subcores
    per_w = n // nw
    ch = min(per_w, 128)
    n_ch = per_w // ch
    mesh = plsc.VectorSubcoreMesh(core_axis_name="c", subcore_axis_name="s")

    @functools.partial(
        pl.kernel,
        mesh=mesh,
        out_type=jax.ShapeDtypeStruct((n, e), table.dtype),
        scratch_types=[
            pltpu.VMEM((ch,), jnp.int32),
            pltpu.VMEM((ch, e), table.dtype),
            pltpu.SemaphoreType.DMA,
        ],
    )
    def gk(table_hbm, idx_hbm, out_hbm, idx_v, rows_v, sem):
        wid = lax.axis_index("s") * info.num_cores + lax.axis_index("c")
        base = wid * per_w
        for c in range(n_ch):
            off = base + c * ch
            pltpu.sync_copy(idx_hbm.at[pl.ds(off, ch)], idx_v)
            pltpu.async_copy(table_hbm.at[idx_v], rows_v, sem).wait()
            pltpu.sync_copy(rows_v, out_hbm.at[pl.ds(off, ch)])

    return gk(table, idx_flat)


_T_BLK = 16


def _gru_body(x_ref, wx_ref, wh_ref, bias_ref, dw_ref, db_ref, out_ref,
              h_ref, hs_ref):
    it = pl.program_id(0)
    u = wh_ref.shape[0]
    tb, b, e = x_ref.shape
    v = dw_ref.shape[1]

    @pl.when(it == 0)
    def _():
        h_ref[...] = jnp.zeros_like(h_ref)

    b_i = bias_ref[0]
    b_r = bias_ref[1]
    b_zr = b_i + jnp.concatenate(
        [b_r[:2 * u], jnp.zeros_like(b_r[2 * u:])])
    b_rh = b_r[2 * u:]
    xp_all = (jnp.dot(x_ref[...].reshape(tb * b, e).astype(jnp.bfloat16),
                      wx_ref[...], preferred_element_type=jnp.float32)
              + b_zr).astype(jnp.bfloat16)
    h = h_ref[...]
    for i in range(tb):
        xp = xp_all[i * b:(i + 1) * b]
        rp = jnp.dot(h.astype(jnp.bfloat16), wh_ref[...],
                     preferred_element_type=jnp.float32)
        z = jax.nn.sigmoid(xp[:, :u] + rp[:, :u])
        r = jax.nn.sigmoid(xp[:, u:2 * u] + rp[:, u:2 * u])
        hh = jnp.tanh(xp[:, 2 * u:] + r * (rp[:, 2 * u:] + b_rh))
        h = z * h + (1.0 - z) * hh
        hs_ref[i * b:(i + 1) * b] = h.astype(jnp.bfloat16)
    h_ref[...] = h
    res = jnp.dot(hs_ref[...], dw_ref[...],
                  preferred_element_type=jnp.float32) + db_ref[...]
    out_ref[...] = res.reshape(tb, b, v).astype(out_ref.dtype)


def _dense_body(a_ref, w_ref, b_ref, o_ref):
    res = (
        jnp.dot(a_ref[...], w_ref[...], preferred_element_type=jnp.float32)
        + b_ref[...]
    )
    o_ref[...] = res.reshape(o_ref.shape)


def kernel(inputs, emb, kernel, rec_kernel, bias, dense_w, dense_b):
    b, s = inputs.shape
    v, e = emb.shape
    u = rec_kernel.shape[0]

    idx = jnp.transpose(inputs.astype(jnp.int32)).reshape(-1)
    x = _sc_gather(emb, idx).reshape(s, b, e)

    tb = _T_BLK
    logits = pl.pallas_call(
        _gru_body,
        grid=(s // tb,),
        in_specs=[
            pl.BlockSpec((tb, b, e), lambda t: (t, 0, 0)),
            pl.BlockSpec((e, 3 * u), lambda t: (0, 0)),
            pl.BlockSpec((u, 3 * u), lambda t: (0, 0)),
            pl.BlockSpec((2, 3 * u), lambda t: (0, 0)),
            pl.BlockSpec((u, v), lambda t: (0, 0)),
            pl.BlockSpec((1, v), lambda t: (0, 0)),
        ],
        out_specs=pl.BlockSpec((tb, b, v), lambda t: (t, 0, 0)),
        out_shape=jax.ShapeDtypeStruct((s, b, v), jnp.float32),
        scratch_shapes=[
            pltpu.VMEM((b, u), jnp.float32),
            pltpu.VMEM((tb * b, u), jnp.bfloat16),
        ],
        compiler_params=pltpu.CompilerParams(
            dimension_semantics=("arbitrary",)),
    )(x, kernel.astype(jnp.bfloat16), rec_kernel.astype(jnp.bfloat16), bias,
      dense_w.astype(jnp.bfloat16), dense_b.reshape(1, v))

    return logits.transpose(1, 0, 2)


# speedup vs baseline: 1.0086x; 1.0086x over previous
import functools

import jax
import jax.numpy as jnp
from jax import lax
from jax.experimental import pallas as pl
from jax.experimental.pallas import tpu as pltpu
from jax.experimental.pallas import tpu_sc as plsc


def _sc_gather(table, idx_flat):
    n = idx_flat.shape[0]
    e = table.shape[1]
    info = plsc.get_sparse_core_info()
    nw = info.num_cores * info.num_subcores
    per_w = n // nw
    ch = min(per_w, 128)
    n_ch = per_w // ch
    mesh = plsc.VectorSubcoreMesh(core_axis_name="c", subcore_axis_name="s")

    @functools.partial(
        pl.kernel,
        mesh=mesh,
        out_type=jax.ShapeDtypeStruct((n, e), table.dtype),
        scratch_types=[
            pltpu.VMEM((ch,), jnp.int32),
            pltpu.VMEM((ch, e), table.dtype),
            pltpu.SemaphoreType.DMA,
        ],
    )
    def gk(table_hbm, idx_hbm, out_hbm, idx_v, rows_v, sem):
        wid = lax.axis_index("s") * info.num_cores + lax.axis_index("c")
        base = wid * per_w
        for c in range(n_ch):
            off = base + c * ch
            pltpu.sync_copy(idx_hbm.at[pl.ds(off, ch)], idx_v)
            pltpu.async_copy(table_hbm.at[idx_v], rows_v, sem).wait()
            pltpu.sync_copy(rows_v, out_hbm.at[pl.ds(off, ch)])

    return gk(table, idx_flat)


_T_BLK = 16


def _gru_body(x_ref, wx_ref, wh_ref, bias_ref, dw_ref, db_ref, out_ref,
              h_ref, hs_ref):
    it = pl.program_id(0)
    u = wh_ref.shape[0]
    tb, b, e = x_ref.shape
    v = dw_ref.shape[1]

    @pl.when(it == 0)
    def _():
        h_ref[...] = jnp.zeros_like(h_ref)

    b_i = bias_ref[0]
    b_r = bias_ref[1]
    b_zr = b_i + jnp.concatenate(
        [b_r[:2 * u], jnp.zeros_like(b_r[2 * u:])])
    b_rh = b_r[2 * u:]
    xp_all = jnp.dot(x_ref[...].reshape(tb * b, e).astype(jnp.bfloat16),
                     wx_ref[...], preferred_element_type=jnp.float32) + b_zr
    h = h_ref[...]
    for i in range(tb):
        xp = xp_all[i * b:(i + 1) * b]
        rp = jnp.dot(h.astype(jnp.bfloat16), wh_ref[...],
                     preferred_element_type=jnp.float32)
        z = jax.nn.sigmoid(xp[:, :u] + rp[:, :u])
        r = jax.nn.sigmoid(xp[:, u:2 * u] + rp[:, u:2 * u])
        hh = jnp.tanh(xp[:, 2 * u:] + r * (rp[:, 2 * u:] + b_rh))
        h = z * h + (1.0 - z) * hh
        hs_ref[i * b:(i + 1) * b] = h.astype(jnp.bfloat16)
    h_ref[...] = h
    res = jnp.dot(hs_ref[...], dw_ref[...],
                  preferred_element_type=jnp.float32) + db_ref[...]
    out_ref[...] = res.reshape(tb, b, v).astype(out_ref.dtype)


def kernel(inputs, emb, kernel, rec_kernel, bias, dense_w, dense_b):
    b, s = inputs.shape
    v, e = emb.shape
    u = rec_kernel.shape[0]

    idx = jnp.transpose(inputs.astype(jnp.int32)).reshape(-1)
    x = _sc_gather(emb, idx).reshape(s, b, e)

    tb = _T_BLK
    logits = pl.pallas_call(
        _gru_body,
        grid=(s // tb,),
        in_specs=[
            pl.BlockSpec((tb, b, e), lambda t: (t, 0, 0)),
            pl.BlockSpec((e, 3 * u), lambda t: (0, 0)),
            pl.BlockSpec((u, 3 * u), lambda t: (0, 0)),
            pl.BlockSpec((2, 3 * u), lambda t: (0, 0)),
            pl.BlockSpec((u, v), lambda t: (0, 0)),
            pl.BlockSpec((1, v), lambda t: (0, 0)),
        ],
        out_specs=pl.BlockSpec((tb, b, v), lambda t: (t, 0, 0)),
        out_shape=jax.ShapeDtypeStruct((s, b, v), jnp.float32),
        scratch_shapes=[
            pltpu.VMEM((b, u), jnp.float32),
            pltpu.VMEM((tb * b, u), jnp.bfloat16),
        ],
        compiler_params=pltpu.CompilerParams(
            dimension_semantics=("arbitrary",)),
    )(x, kernel.astype(jnp.bfloat16), rec_kernel.astype(jnp.bfloat16), bias,
      dense_w.astype(jnp.bfloat16), dense_b.reshape(1, v))

    return logits.transpose(1, 0, 2)
